# baseline (speedup 1.0000x reference)
import functools

import jax
import jax.numpy as jnp
from jax import lax
from jax.experimental import pallas as pl
from jax.experimental.pallas import tpu as pltpu
from jax.experimental.pallas import tpu_sc as plsc

_TOPIC_DIM = 64
_SUBTOPIC_DIM = 32
_CHUNK = 128


@functools.lru_cache(maxsize=None)
def _make_kernel(B: int):
    info = plsc.get_sparse_core_info()
    NC, NS = info.num_cores, info.num_subcores
    NW = NC * NS
    assert B % (NW * _CHUNK) == 0
    bpw = B // NW
    nchunks = bpw // _CHUNK
    out_dim = _TOPIC_DIM + _SUBTOPIC_DIM

    mesh = plsc.VectorSubcoreMesh(core_axis_name="c", subcore_axis_name="s")

    @functools.partial(
        pl.kernel,
        mesh=mesh,
        compiler_params=pltpu.CompilerParams(use_tc_tiling_on_sc=False),
        out_type=jax.ShapeDtypeStruct((B, out_dim), jnp.float32),
        scratch_types=[
            pltpu.VMEM((nchunks, _CHUNK), jnp.int32),
            pltpu.VMEM((nchunks, _CHUNK), jnp.int32),
            pltpu.VMEM((bpw, _TOPIC_DIM), jnp.float32),
            pltpu.VMEM((bpw, _SUBTOPIC_DIM), jnp.float32),
            pltpu.SemaphoreType.DMA,
        ],
    )
    def k(topic_hbm, subtopic_hbm, title_hbm, sub_hbm, out_hbm,
          tidx_v, sidx_v, trows_v, srows_v, sem):
        wid = lax.axis_index("s") * NC + lax.axis_index("c")
        base = wid * bpw
        pltpu.sync_copy(topic_hbm.at[wid], tidx_v)
        pltpu.sync_copy(subtopic_hbm.at[wid], sidx_v)
        copies = []
        for j in range(nchunks):
            copies.append(pltpu.async_copy(
                title_hbm.at[tidx_v.at[j]],
                trows_v.at[pl.ds(j * _CHUNK, _CHUNK)], sem))
            copies.append(pltpu.async_copy(
                sub_hbm.at[sidx_v.at[j]],
                srows_v.at[pl.ds(j * _CHUNK, _CHUNK)], sem))
        for c in copies:
            c.wait()
        pltpu.sync_copy(trows_v,
                        out_hbm.at[pl.ds(base, bpw), pl.ds(0, _TOPIC_DIM)])
        pltpu.sync_copy(srows_v,
                        out_hbm.at[pl.ds(base, bpw),
                                   pl.ds(_TOPIC_DIM, _SUBTOPIC_DIM)])

    return k


def kernel(topic, subtopic, title_embed, subtopic_embed):
    B = topic.shape[0]
    info = plsc.get_sparse_core_info()
    NW = info.num_cores * info.num_subcores
    t = topic.astype(jnp.int32).reshape(NW, -1, _CHUNK)
    s = subtopic.astype(jnp.int32).reshape(NW, -1, _CHUNK)
    return _make_kernel(B)(t, s, title_embed, subtopic_embed)
